# baseline (speedup 1.0000x reference)
import functools

import jax
import jax.numpy as jnp
from jax import lax
from jax.experimental import pallas as pl
from jax.experimental.pallas import tpu as pltpu
from jax.experimental.pallas import tpu_sc as plsc

_NUM_CORES = 2
_NUM_SUBCORES = 16
_NW = _NUM_CORES * _NUM_SUBCORES

_D = 16
_B = 16384 * 50 * 4
_BPW = _B // _NW
_C = 1600
_NBUF = 1
_NGROUP = _BPW // (_C * _NBUF)

_mesh = plsc.VectorSubcoreMesh(core_axis_name="c", subcore_axis_name="s")


@functools.partial(
    pl.kernel,
    out_type=jax.ShapeDtypeStruct((_B, _D), jnp.float32),
    mesh=_mesh,
    scratch_types=[
        pltpu.VMEM((_NBUF, _C), jnp.int32),
        pltpu.VMEM((_NBUF, _C, _D), jnp.float32),
        pltpu.SemaphoreType.DMA,
        pltpu.SemaphoreType.DMA,
    ],
    compiler_params=pltpu.CompilerParams(use_tc_tiling_on_sc=False),
)
def _sc_gather(idx_hbm, table_hbm, out_hbm, idx_v, rows_v, sem_i, sem_g):
    wid = lax.axis_index("s") * _NUM_CORES + lax.axis_index("c")
    base0 = wid * _BPW

    def body(g, carry):
        gbase = base0 + g * _C * _NBUF
        copies = []
        for b in range(_NBUF):
            base = gbase + b * _C
            copies.append(pltpu.async_copy(
                idx_hbm.at[pl.ds(base, _C)], idx_v.at[b], sem_i))
        gathers = []
        for b in range(_NBUF):
            copies[b].wait()
            gathers.append(pltpu.async_copy(
                table_hbm.at[idx_v.at[b]], rows_v.at[b], sem_g))
        for b in range(_NBUF):
            gathers[b].wait()
        return carry

    lax.fori_loop(0, _NGROUP, body, 0)


def kernel(token_ids, embedding_matrix):
    n, s = token_ids.shape
    flat = token_ids.reshape(-1).astype(jnp.int32)
    eidx = (flat[:, None] * 4 + jnp.arange(4, dtype=jnp.int32)[None, :]).reshape(-1)
    tab16 = embedding_matrix.reshape(-1, 16)
    out = _sc_gather(eidx, tab16)
    return out.reshape(n, s, 64)


# speedup vs baseline: 1.4438x; 1.4438x over previous
import functools

import jax
import jax.numpy as jnp
from jax import lax
from jax.experimental import pallas as pl
from jax.experimental.pallas import tpu as pltpu
from jax.experimental.pallas import tpu_sc as plsc

_NUM_CORES = 2
_NUM_SUBCORES = 16
_NW = _NUM_CORES * _NUM_SUBCORES

_D = 64
_B = 16384 * 50
_BPW = _B // _NW
_C = 400
_NBUF = 4
_NGROUP = _BPW // (_C * _NBUF)

_mesh = plsc.VectorSubcoreMesh(core_axis_name="c", subcore_axis_name="s")


@functools.partial(
    pl.kernel,
    out_type=jax.ShapeDtypeStruct((_B, _D), jnp.float32),
    mesh=_mesh,
    scratch_types=[
        pltpu.VMEM((_NBUF, _C), jnp.int32),
        pltpu.VMEM((_NBUF, _C, _D), jnp.float32),
        pltpu.SemaphoreType.DMA,
        pltpu.SemaphoreType.DMA,
        pltpu.SemaphoreType.DMA,
    ],
    compiler_params=pltpu.CompilerParams(use_tc_tiling_on_sc=False),
)
def _sc_gather(idx_hbm, table_hbm, out_hbm, idx_v, rows_v, sem_i, sem_g, sem_o):
    wid = lax.axis_index("s") * _NUM_CORES + lax.axis_index("c")
    base0 = wid * _BPW

    def body(g, carry):
        gbase = base0 + g * _C * _NBUF
        copies = []
        for b in range(_NBUF):
            base = gbase + b * _C
            copies.append(pltpu.async_copy(
                idx_hbm.at[pl.ds(base, _C)], idx_v.at[b], sem_i))
        gathers = []
        for b in range(_NBUF):
            copies[b].wait()
            gathers.append(pltpu.async_copy(
                table_hbm.at[idx_v.at[b]], rows_v.at[b], sem_g))
        stores = []
        for b in range(_NBUF):
            base = gbase + b * _C
            gathers[b].wait()
            stores.append(pltpu.async_copy(
                rows_v.at[b], out_hbm.at[pl.ds(base, _C)], sem_o))
        for b in range(_NBUF):
            stores[b].wait()
        return carry

    lax.fori_loop(0, _NGROUP, body, 0)


def kernel(token_ids, embedding_matrix):
    n, s = token_ids.shape
    flat_ids = token_ids.reshape(-1).astype(jnp.int32)
    out = _sc_gather(flat_ids, embedding_matrix)
    return out.reshape(n, s, _D)


# speedup vs baseline: 1.9488x; 1.3498x over previous
import functools

import jax
import jax.numpy as jnp
from jax import lax
from jax.experimental import pallas as pl
from jax.experimental.pallas import tpu as pltpu
from jax.experimental.pallas import tpu_sc as plsc

_NUM_CORES = 2
_NUM_SUBCORES = 16
_NW = _NUM_CORES * _NUM_SUBCORES

_D = 64
_N = 16384
_S = 50
_SP = 56
_LP = 128
_B = _N * _S
_NPW = _N // _NW
_BPW = _B // _NW
_CN = 8
_C = _CN * _S
_NBUF = 4
_NGROUP = _NPW // (_CN * _NBUF)

_mesh = plsc.VectorSubcoreMesh(core_axis_name="c", subcore_axis_name="s")


@functools.partial(
    pl.kernel,
    out_type=jax.ShapeDtypeStruct((_N * _SP, _LP), jnp.float32),
    mesh=_mesh,
    scratch_types=[
        pltpu.VMEM((_NBUF, _C), jnp.int32),
        pltpu.VMEM((_NBUF, _C, _D), jnp.float32),
        pltpu.SemaphoreType.DMA,
        pltpu.SemaphoreType.DMA,
        pltpu.SemaphoreType.DMA,
    ],
    compiler_params=pltpu.CompilerParams(use_tc_tiling_on_sc=False),
)
def _sc_gather(idx_hbm, table_hbm, out_hbm, idx_v, rows_v, sem_i, sem_g, sem_o):
    wid = lax.axis_index("s") * _NUM_CORES + lax.axis_index("c")
    tok0 = wid * _BPW
    seq0 = wid * _NPW

    def body(g, carry):
        gtok = tok0 + g * _C * _NBUF
        gseq = seq0 + g * _CN * _NBUF
        copies = []
        for b in range(_NBUF):
            base = gtok + b * _C
            copies.append(pltpu.async_copy(
                idx_hbm.at[pl.ds(base, _C)], idx_v.at[b], sem_i))
        gathers = []
        for b in range(_NBUF):
            copies[b].wait()
            gathers.append(pltpu.async_copy(
                table_hbm.at[idx_v.at[b]], rows_v.at[b], sem_g))
        stores = []
        for b in range(_NBUF):
            gathers[b].wait()
            for j in range(_CN):
                n = gseq + b * _CN + j
                stores.append(pltpu.async_copy(
                    rows_v.at[b, pl.ds(j * _S, _S), :],
                    out_hbm.at[pl.ds(n * _SP, _S), pl.ds(0, _D)],
                    sem_o))
        for st in stores:
            st.wait()
        return carry

    lax.fori_loop(0, _NGROUP, body, 0)


def kernel(token_ids, embedding_matrix):
    flat_ids = token_ids.reshape(-1).astype(jnp.int32)
    out = _sc_gather(flat_ids, embedding_matrix)
    return out.reshape(_N, _SP, _LP)[:, :_S, :_D]
